# baseline (speedup 1.0000x reference)
import jax
import jax.numpy as jnp
from jax.experimental import pallas as pl
from jax.experimental.pallas import tpu as pltpu

N_BINS = 15
ROWS_PER_BLOCK = 1000
N_STREAMS = 4


def _partials_kernel(lo_ref, hi_ref, *refs):
    i = pl.program_id(0)
    out_ref = refs[-1]

    @pl.when(i == 0)
    def _init():
        out_ref[...] = jnp.zeros_like(out_ref)

    lo = lo_ref[...]
    hi = hi_ref[...]

    for s in range(N_STREAMS):
        x = refs[s][...]
        c = x.shape[1]
        lab = refs[N_STREAMS + s][...].astype(jnp.float32)
        conf = jnp.max(x, axis=1, keepdims=True)
        col = jax.lax.broadcasted_iota(jnp.int32, x.shape, 1).astype(jnp.float32)
        pred = jnp.min(jnp.where(x == conf, col, jnp.float32(c)), axis=1,
                       keepdims=True)
        acc = (pred == lab).astype(jnp.float32)
        onehot = ((conf > lo) & (conf <= hi)).astype(jnp.float32)

        out_ref[0:1, :] += jnp.sum(onehot, axis=0, keepdims=True)
        out_ref[1:2, :] += jnp.sum(onehot * acc, axis=0, keepdims=True)
        out_ref[2:3, :] += jnp.sum(onehot * conf, axis=0, keepdims=True)


def _finish_kernel(part_ref, out_ref):
    num = part_ref[0:1, :]
    sacc = part_ref[1:2, :]
    sconf = part_ref[2:3, :]
    safe_n = jnp.maximum(num, 1.0)
    acc_bin = sacc / safe_n
    conf_bin = sconf / safe_n
    has = num > 0.0
    ece = jnp.sum(jnp.where(has, jnp.abs(conf_bin - acc_bin) * num, 0.0))
    out_ref[0:1, :] = jnp.full_like(num, ece)
    out_ref[1:2, :] = jnp.where(has, acc_bin * num, 0.0)
    out_ref[2:3, :] = jnp.where(has, num, 0.0)


def kernel(probs, labels, mode):
    n, c = probs.shape
    r = ROWS_PER_BLOCK
    nblk = n // (r * N_STREAMS)

    bb = jnp.linspace(0.0, 1.0, N_BINS + 1)
    lo = jnp.full((1, 128), 2.0, dtype=jnp.float32).at[0, :N_BINS].set(bb[:-1])
    hi = jnp.full((1, 128), -1.0, dtype=jnp.float32).at[0, :N_BINS].set(bb[1:])
    labels2 = labels.reshape(n, 1)

    def make_spec(stream, shape):
        return pl.BlockSpec(shape, lambda i, s=stream: (s * nblk + i, 0))

    partials = pl.pallas_call(
        _partials_kernel,
        grid=(nblk,),
        in_specs=(
            [pl.BlockSpec((1, 128), lambda i: (0, 0))] * 2
            + [make_spec(s, (r, c)) for s in range(N_STREAMS)]
            + [make_spec(s, (r, 1)) for s in range(N_STREAMS)]
        ),
        out_specs=pl.BlockSpec((8, 128), lambda i: (0, 0)),
        out_shape=jax.ShapeDtypeStruct((8, 128), jnp.float32),
    )(lo, hi, *([probs] * N_STREAMS), *([labels2] * N_STREAMS))

    out = pl.pallas_call(
        _finish_kernel,
        out_shape=jax.ShapeDtypeStruct((8, 128), jnp.float32),
    )(partials)

    ece = out[0, 0:1]
    correct = out[1, 0:N_BINS]
    num = out[2, 0:N_BINS]
    return (ece, correct, num)


# speedup vs baseline: 1.0019x; 1.0019x over previous
import jax
import jax.numpy as jnp
from jax.experimental import pallas as pl
from jax.experimental.pallas import tpu as pltpu

N_BINS = 15
ROWS_PER_BLOCK = 4000
N_STREAMS = 1


def _partials_kernel(lo_ref, hi_ref, *refs):
    i = pl.program_id(0)
    out_ref = refs[-1]

    @pl.when(i == 0)
    def _init():
        out_ref[...] = jnp.zeros_like(out_ref)

    lo = lo_ref[...]
    hi = hi_ref[...]

    for s in range(N_STREAMS):
        x = refs[s][...]
        c = x.shape[1]
        lab = refs[N_STREAMS + s][...].astype(jnp.float32)
        conf = jnp.max(x, axis=1, keepdims=True)
        col = jax.lax.broadcasted_iota(jnp.int32, x.shape, 1).astype(jnp.float32)
        pred = jnp.min(jnp.where(x == conf, col, jnp.float32(c)), axis=1,
                       keepdims=True)
        acc = (pred == lab).astype(jnp.float32)
        onehot = ((conf > lo) & (conf <= hi)).astype(jnp.float32)

        out_ref[0:1, :] += jnp.sum(onehot, axis=0, keepdims=True)
        out_ref[1:2, :] += jnp.sum(onehot * acc, axis=0, keepdims=True)
        out_ref[2:3, :] += jnp.sum(onehot * conf, axis=0, keepdims=True)


def _finish_kernel(part_ref, out_ref):
    num = part_ref[0:1, :]
    sacc = part_ref[1:2, :]
    sconf = part_ref[2:3, :]
    safe_n = jnp.maximum(num, 1.0)
    acc_bin = sacc / safe_n
    conf_bin = sconf / safe_n
    has = num > 0.0
    ece = jnp.sum(jnp.where(has, jnp.abs(conf_bin - acc_bin) * num, 0.0))
    out_ref[0:1, :] = jnp.full_like(num, ece)
    out_ref[1:2, :] = jnp.where(has, acc_bin * num, 0.0)
    out_ref[2:3, :] = jnp.where(has, num, 0.0)


def kernel(probs, labels, mode):
    n, c = probs.shape
    r = ROWS_PER_BLOCK
    nblk = n // (r * N_STREAMS)

    bb = jnp.linspace(0.0, 1.0, N_BINS + 1)
    lo = jnp.full((1, 128), 2.0, dtype=jnp.float32).at[0, :N_BINS].set(bb[:-1])
    hi = jnp.full((1, 128), -1.0, dtype=jnp.float32).at[0, :N_BINS].set(bb[1:])
    labels2 = labels.reshape(n, 1)

    def make_spec(stream, shape):
        return pl.BlockSpec(shape, lambda i, s=stream: (s * nblk + i, 0))

    partials = pl.pallas_call(
        _partials_kernel,
        grid=(nblk,),
        in_specs=(
            [pl.BlockSpec((1, 128), lambda i: (0, 0))] * 2
            + [make_spec(s, (r, c)) for s in range(N_STREAMS)]
            + [make_spec(s, (r, 1)) for s in range(N_STREAMS)]
        ),
        out_specs=pl.BlockSpec((8, 128), lambda i: (0, 0)),
        out_shape=jax.ShapeDtypeStruct((8, 128), jnp.float32),
    )(lo, hi, *([probs] * N_STREAMS), *([labels2] * N_STREAMS))

    out = pl.pallas_call(
        _finish_kernel,
        out_shape=jax.ShapeDtypeStruct((8, 128), jnp.float32),
    )(partials)

    ece = out[0, 0:1]
    correct = out[1, 0:N_BINS]
    num = out[2, 0:N_BINS]
    return (ece, correct, num)


# speedup vs baseline: 1.0065x; 1.0046x over previous
import jax
import jax.numpy as jnp
from jax.experimental import pallas as pl
from jax.experimental.pallas import tpu as pltpu

N_BINS = 15
ROWS_PER_BLOCK = 2000
N_STREAMS = 1


def _partials_kernel(lo_ref, hi_ref, *refs):
    i = pl.program_id(0)
    out_ref = refs[-1]

    @pl.when(i == 0)
    def _init():
        out_ref[...] = jnp.zeros_like(out_ref)

    lo = lo_ref[...]
    hi = hi_ref[...]

    for s in range(N_STREAMS):
        x = refs[s][...]
        c = x.shape[1]
        lab = refs[N_STREAMS + s][...].astype(jnp.float32)
        conf = jnp.max(x, axis=1, keepdims=True)
        col = jax.lax.broadcasted_iota(jnp.int32, x.shape, 1).astype(jnp.float32)
        pred = jnp.min(jnp.where(x == conf, col, jnp.float32(c)), axis=1,
                       keepdims=True)
        acc = (pred == lab).astype(jnp.float32)
        onehot = ((conf > lo) & (conf <= hi)).astype(jnp.float32)

        out_ref[0:1, :] += jnp.sum(onehot, axis=0, keepdims=True)
        out_ref[1:2, :] += jnp.sum(onehot * acc, axis=0, keepdims=True)
        out_ref[2:3, :] += jnp.sum(onehot * conf, axis=0, keepdims=True)


def _finish_kernel(part_ref, out_ref):
    num = part_ref[0:1, :]
    sacc = part_ref[1:2, :]
    sconf = part_ref[2:3, :]
    safe_n = jnp.maximum(num, 1.0)
    acc_bin = sacc / safe_n
    conf_bin = sconf / safe_n
    has = num > 0.0
    ece = jnp.sum(jnp.where(has, jnp.abs(conf_bin - acc_bin) * num, 0.0))
    out_ref[0:1, :] = jnp.full_like(num, ece)
    out_ref[1:2, :] = jnp.where(has, acc_bin * num, 0.0)
    out_ref[2:3, :] = jnp.where(has, num, 0.0)


def kernel(probs, labels, mode):
    n, c = probs.shape
    r = ROWS_PER_BLOCK
    nblk = n // (r * N_STREAMS)

    bb = jnp.linspace(0.0, 1.0, N_BINS + 1)
    lo = jnp.full((1, 128), 2.0, dtype=jnp.float32).at[0, :N_BINS].set(bb[:-1])
    hi = jnp.full((1, 128), -1.0, dtype=jnp.float32).at[0, :N_BINS].set(bb[1:])
    labels2 = labels.reshape(n, 1)

    def make_spec(stream, shape):
        return pl.BlockSpec(shape, lambda i, s=stream: (s * nblk + i, 0))

    partials = pl.pallas_call(
        _partials_kernel,
        grid=(nblk,),
        in_specs=(
            [pl.BlockSpec((1, 128), lambda i: (0, 0))] * 2
            + [make_spec(s, (r, c)) for s in range(N_STREAMS)]
            + [make_spec(s, (r, 1)) for s in range(N_STREAMS)]
        ),
        out_specs=pl.BlockSpec((8, 128), lambda i: (0, 0)),
        out_shape=jax.ShapeDtypeStruct((8, 128), jnp.float32),
    )(lo, hi, *([probs] * N_STREAMS), *([labels2] * N_STREAMS))

    out = pl.pallas_call(
        _finish_kernel,
        out_shape=jax.ShapeDtypeStruct((8, 128), jnp.float32),
    )(partials)

    ece = out[0, 0:1]
    correct = out[1, 0:N_BINS]
    num = out[2, 0:N_BINS]
    return (ece, correct, num)
